# Initial kernel scaffold; baseline (speedup 1.0000x reference)
import jax
import jax.numpy as jnp
from jax.experimental import pallas as pl


def kernel(x, edge_index, edge_weight, edge_label_index, W1, b1, W2, b2, Wl, bl):
    raise NotImplementedError("write your pallas kernel here")


# speedup vs baseline: 4.9047x; 4.9047x over previous
import functools

import jax
import jax.numpy as jnp
from jax import lax
from jax.experimental import pallas as pl
from jax.experimental.pallas import tpu as pltpu
from jax.experimental.pallas import tpu_sc as plsc

N = 10000
C = 128
E = 320000
LBL = 200000

NC, NS, LANES = 2, 16, 16
NW = NC * NS

EP = 327680
ECH = 80
LP = 204800
LCH = 50

NPAD = 10240

_MESH = plsc.VectorSubcoreMesh(core_axis_name="c", subcore_axis_name="s")


def _wid():
    return lax.axis_index("s") * NC + lax.axis_index("c")


@functools.partial(
    pl.kernel,
    out_type=jax.ShapeDtypeStruct((NC, NPAD), jnp.float32),
    mesh=_MESH,
    scratch_types=[
        pltpu.VMEM((128,), jnp.int32),
        pltpu.VMEM((128,), jnp.float32),
        pltpu.VMEM((640,), jnp.float32),
        pltpu.VMEM_SHARED((NPAD,), jnp.float32),
    ],
)
def _sc_deg(dst2, ew2, deg_out, didx_v, ew_v, zb, deg_sh):
    cid = lax.axis_index("c")
    sid = lax.axis_index("s")
    wid = _wid()
    zero16 = jnp.zeros((16,), jnp.float32)

    def zloop(i, carry):
        zb[pl.ds(i * 16, 16)] = zero16
        return carry

    lax.fori_loop(0, 40, zloop, None)
    pltpu.sync_copy(zb, deg_sh.at[pl.ds(sid * 640, 640)])
    plsc.subcore_barrier()

    def body(i, carry):
        ch = wid * ECH + i
        pltpu.sync_copy(dst2.at[pl.ds(ch * 128, 128)], didx_v)
        pltpu.sync_copy(ew2.at[pl.ds(ch * 128, 128)], ew_v)
        pltpu.sync_copy(ew_v, deg_sh.at[didx_v], add=True)
        return carry

    lax.fori_loop(0, ECH, body, None)
    plsc.subcore_barrier()
    pltpu.sync_copy(deg_sh.at[pl.ds(sid * 640, 640)],
                    deg_out.at[cid, pl.ds(sid * 640, 640)])


def _splat(v, b):
    return jnp.broadcast_to(lax.slice(v, (b,), (b + 1,)), (16,))


_GDN = lax.GatherDimensionNumbers(
    offset_dims=(), collapsed_slice_dims=(0,), start_index_map=(0,))


def _shuffle(v, idx):
    return lax.gather(v, idx[:, None], _GDN, (1,),
                      mode=lax.GatherScatterMode.PROMISE_IN_BOUNDS)


def _lane_sum(v, iota16):
    for sh in (8, 4, 2, 1):
        v = v + _shuffle(v, iota16 ^ sh)
    return v


def _scale_rows(rows, nv, g, iota16):
    for b in range(16):
        sb = _splat(nv, b)
        r = g * 16 + b
        for t in range(8):
            rows[r, pl.ds(t * 16, 16)] = rows[r, pl.ds(t * 16, 16)] * sb


@functools.partial(
    pl.kernel,
    out_type=(jax.ShapeDtypeStruct((NC, N, C), jnp.float32),
              jax.ShapeDtypeStruct((EP,), jnp.float32)),
    mesh=_MESH,
    scratch_types=[
        pltpu.VMEM_SHARED((NPAD,), jnp.float32),
        pltpu.VMEM((128,), jnp.int32),
        pltpu.VMEM((128,), jnp.int32),
        pltpu.VMEM((128,), jnp.float32),
        pltpu.VMEM((128,), jnp.float32),
        pltpu.VMEM((128,), jnp.float32),
        pltpu.VMEM((128,), jnp.float32),
        pltpu.VMEM((128, C), jnp.float32),
        pltpu.VMEM((16, C), jnp.float32),
        pltpu.VMEM_SHARED((N, C), jnp.float32),
        pltpu.SemaphoreType.DMA,
    ],
)
def _sc_msg1(src2, dst2, ew2, dis_hbm, h_hbm, acc_out, norm_out,
             dis_sh, sidx, didx, ewv, normv, dis_s, dis_d, rows, zb, acc_sh, sem):
    cid = lax.axis_index("c")
    sid = lax.axis_index("s")
    wid = _wid()
    iota16 = lax.iota(jnp.int32, 16)
    zero16 = jnp.zeros((16,), jnp.float32)

    @pl.when(sid == 0)
    def _():
        pltpu.sync_copy(dis_hbm, dis_sh)

    for r in range(16):
        for t in range(8):
            zb[r, pl.ds(t * 16, 16)] = zero16

    def zloop(k, carry):
        pltpu.sync_copy(zb, acc_sh.at[pl.ds(sid * 624 + k * 16, 16)])
        return carry

    lax.fori_loop(0, 39, zloop, None)

    @pl.when(sid == 15)
    def _():
        pltpu.sync_copy(zb, acc_sh.at[pl.ds(9984, 16)])

    plsc.subcore_barrier()

    def body(i, carry):
        ch = wid * ECH + i
        pltpu.sync_copy(src2.at[pl.ds(ch * 128, 128)], sidx)
        pltpu.sync_copy(dst2.at[pl.ds(ch * 128, 128)], didx)
        pltpu.sync_copy(ew2.at[pl.ds(ch * 128, 128)], ewv)
        pltpu.sync_copy(dis_sh.at[sidx], dis_s)
        pltpu.sync_copy(dis_sh.at[didx], dis_d)
        pltpu.async_copy(h_hbm.at[sidx], rows, sem).wait()
        for g in range(8):
            ev = ewv[pl.ds(g * 16, 16)]
            nv = dis_s[pl.ds(g * 16, 16)] * ev * dis_d[pl.ds(g * 16, 16)]
            normv[pl.ds(g * 16, 16)] = nv
            _scale_rows(rows, nv, g, iota16)
        pltpu.sync_copy(rows, acc_sh.at[didx], add=True)
        pltpu.sync_copy(normv, norm_out.at[pl.ds(ch * 128, 128)])
        return carry

    lax.fori_loop(0, ECH, body, None)
    plsc.subcore_barrier()
    pltpu.sync_copy(acc_sh.at[pl.ds(sid * 624, 624)],
                    acc_out.at[cid, pl.ds(sid * 624, 624)])

    @pl.when(sid == 15)
    def _():
        pltpu.sync_copy(acc_sh.at[pl.ds(9984, 16)],
                        acc_out.at[cid, pl.ds(9984, 16)])


@functools.partial(
    pl.kernel,
    out_type=jax.ShapeDtypeStruct((NC, N, C), jnp.float32),
    mesh=_MESH,
    scratch_types=[
        pltpu.VMEM((128,), jnp.int32),
        pltpu.VMEM((128,), jnp.int32),
        pltpu.VMEM((128,), jnp.float32),
        pltpu.VMEM((128, C), jnp.float32),
        pltpu.VMEM((16, C), jnp.float32),
        pltpu.VMEM_SHARED((N, C), jnp.float32),
        pltpu.SemaphoreType.DMA,
    ],
)
def _sc_msg2(src2, dst2, norm2, h_hbm, acc_out,
             sidx, didx, normv, rows, zb, acc_sh, sem):
    cid = lax.axis_index("c")
    sid = lax.axis_index("s")
    wid = _wid()
    iota16 = lax.iota(jnp.int32, 16)
    zero16 = jnp.zeros((16,), jnp.float32)

    for r in range(16):
        for t in range(8):
            zb[r, pl.ds(t * 16, 16)] = zero16

    def zloop(k, carry):
        pltpu.sync_copy(zb, acc_sh.at[pl.ds(sid * 624 + k * 16, 16)])
        return carry

    lax.fori_loop(0, 39, zloop, None)

    @pl.when(sid == 15)
    def _():
        pltpu.sync_copy(zb, acc_sh.at[pl.ds(9984, 16)])

    plsc.subcore_barrier()

    def body(i, carry):
        ch = wid * ECH + i
        pltpu.sync_copy(src2.at[pl.ds(ch * 128, 128)], sidx)
        pltpu.sync_copy(dst2.at[pl.ds(ch * 128, 128)], didx)
        pltpu.sync_copy(norm2.at[pl.ds(ch * 128, 128)], normv)
        pltpu.async_copy(h_hbm.at[sidx], rows, sem).wait()
        for g in range(8):
            nv = normv[pl.ds(g * 16, 16)]
            _scale_rows(rows, nv, g, iota16)
        pltpu.sync_copy(rows, acc_sh.at[didx], add=True)
        return carry

    lax.fori_loop(0, ECH, body, None)
    plsc.subcore_barrier()
    pltpu.sync_copy(acc_sh.at[pl.ds(sid * 624, 624)],
                    acc_out.at[cid, pl.ds(sid * 624, 624)])

    @pl.when(sid == 15)
    def _():
        pltpu.sync_copy(acc_sh.at[pl.ds(9984, 16)],
                        acc_out.at[cid, pl.ds(9984, 16)])


@functools.partial(
    pl.kernel,
    out_type=jax.ShapeDtypeStruct((LP,), jnp.float32),
    mesh=_MESH,
    scratch_types=[
        pltpu.VMEM((128,), jnp.int32),
        pltpu.VMEM((128,), jnp.int32),
        pltpu.VMEM((128, C), jnp.float32),
        pltpu.VMEM((128, C), jnp.float32),
        pltpu.VMEM((128,), jnp.float32),
        pltpu.VMEM((16,), jnp.float32),
        pltpu.SemaphoreType.DMA,
        pltpu.SemaphoreType.DMA,
    ],
)
def _sc_dot(sl2, dl2, a_hbm, c_hbm, bls_hbm, res_out,
            sidx, didx, rs, rd, resv, blsv, sem, sem2):
    wid = _wid()
    iota16 = lax.iota(jnp.int32, 16)
    pltpu.sync_copy(bls_hbm, blsv)
    bv = blsv[...]

    def body(i, carry):
        ch = wid * LCH + i
        pltpu.sync_copy(sl2.at[pl.ds(ch * 128, 128)], sidx)
        pltpu.sync_copy(dl2.at[pl.ds(ch * 128, 128)], didx)
        cp1 = pltpu.async_copy(a_hbm.at[sidx], rs, sem)
        cp2 = pltpu.async_copy(c_hbm.at[didx], rd, sem2)
        cp1.wait()
        cp2.wait()
        for g in range(8):
            res_v = jnp.zeros((16,), jnp.float32)
            for b in range(16):
                e = g * 16 + b
                acc = rs[e, pl.ds(0, 16)] * rd[e, pl.ds(0, 16)]
                for t in range(1, 8):
                    acc = acc + rs[e, pl.ds(t * 16, 16)] * rd[e, pl.ds(t * 16, 16)]
                res_v = jnp.where(iota16 == b, _lane_sum(acc, iota16), res_v)
            resv[pl.ds(g * 16, 16)] = res_v + bv
        pltpu.sync_copy(resv, res_out.at[pl.ds(ch * 128, 128)])
        return carry

    lax.fori_loop(0, LCH, body, None)


def _tc_lin1_body(x_ref, w_ref, degp_ref, h_ref, dis_ref, dis2_ref):
    deg = degp_ref[0, :] + degp_ref[1, :] + 1.0
    dis = lax.rsqrt(deg)
    dis_ref[...] = dis[None, :]
    dis2_ref[...] = (dis * dis)[None, :]
    h_ref[...] = jnp.dot(x_ref[...], w_ref[...],
                         preferred_element_type=jnp.float32)


def _tc_lin1(x, w1, degp):
    return pl.pallas_call(
        _tc_lin1_body,
        out_shape=(jax.ShapeDtypeStruct((N, C), jnp.float32),
                   jax.ShapeDtypeStruct((1, NPAD), jnp.float32),
                   jax.ShapeDtypeStruct((1, NPAD), jnp.float32)),
    )(x, w1, degp)


def _tc_lin2_body(accp_ref, h_ref, dis2_ref, b_ref, w_ref, h2_ref):
    out = accp_ref[0] + accp_ref[1] + h_ref[...] * dis2_ref[...] + b_ref[...]
    out = jnp.maximum(out, 0.0)
    h2_ref[...] = jnp.dot(out, w_ref[...], preferred_element_type=jnp.float32)


def _tc_lin2(accp, h1, dis2c, b1r, w2):
    return pl.pallas_call(
        _tc_lin2_body,
        out_shape=jax.ShapeDtypeStruct((N, C), jnp.float32),
    )(accp, h1, dis2c, b1r, w2)


def _tc_fin_body(accp_ref, h_ref, dis2_ref, b_ref, wl_ref, bl_ref,
                 a_ref, c_ref, bls_ref):
    out = accp_ref[0] + accp_ref[1] + h_ref[...] * dis2_ref[...] + b_ref[...]
    out = jnp.maximum(out, 0.0)
    a_ref[...] = out
    wl = jnp.sum(wl_ref[...], axis=1)
    c_ref[...] = out * wl[None, :]
    bls_ref[...] = jnp.sum(bl_ref[...]) * jnp.ones((1, C), jnp.float32)


def _tc_fin(accp, h2, dis2c, b2r, wl, blr):
    return pl.pallas_call(
        _tc_fin_body,
        out_shape=(jax.ShapeDtypeStruct((N, C), jnp.float32),
                   jax.ShapeDtypeStruct((N, C), jnp.float32),
                   jax.ShapeDtypeStruct((1, C), jnp.float32)),
    )(accp, h2, dis2c, b2r, wl, blr)


def kernel(x, edge_index, edge_weight, edge_label_index, W1, b1, W2, b2, Wl, bl):
    src = edge_index[0].astype(jnp.int32)
    dst = edge_index[1].astype(jnp.int32)
    ew = edge_weight.astype(jnp.float32)
    ls = edge_label_index[0].astype(jnp.int32)
    ld = edge_label_index[1].astype(jnp.int32)

    zpi = jnp.zeros((EP - E,), jnp.int32)
    src2 = jnp.concatenate([src, zpi])
    dst2 = jnp.concatenate([dst, zpi])
    ew2 = jnp.concatenate([ew, jnp.zeros((EP - E,), jnp.float32)])
    zpl = jnp.zeros((LP - LBL,), jnp.int32)
    sl2 = jnp.concatenate([ls, zpl])
    dl2 = jnp.concatenate([ld, zpl])

    degp = _sc_deg(dst2, ew2)
    h1, dis_r, dis2_r = _tc_lin1(x, W1, degp)
    dis_f = dis_r.reshape(NPAD)
    dis2c = dis2_r[0, :N][:, None]

    acc1, norm2 = _sc_msg1(src2, dst2, ew2, dis_f, h1)
    h2 = _tc_lin2(acc1, h1, dis2c, b1.reshape(1, C), W2)
    acc2 = _sc_msg2(src2, dst2, norm2, h2)
    a, c, bls = _tc_fin(acc2, h2, dis2c, b2.reshape(1, C), Wl,
                        bl.reshape(1, C))
    res = _sc_dot(sl2, dl2, a, c, bls.reshape(C)[:16])
    return res[:LBL]
